# baseline (speedup 1.0000x reference)
import functools

import jax
import jax.numpy as jnp
from jax import lax
from jax.experimental import pallas as pl
from jax.experimental.pallas import tpu as pltpu
from jax.experimental.pallas import tpu_sc as plsc

VOCAB = 1000
EMB_DIM = 64
BATCH = 4096
HIST = 50

BBLK = 128
NBT = BATCH // BBLK
NBLOCKS = HIST * NBT
ETILES = EMB_DIM // 8
TSTRIDE = VOCAB


def _make_kernel():
    info = plsc.get_sparse_core_info()
    nc, ns, nl = info.num_cores, info.num_subcores, info.num_lanes
    nw = nc * ns
    bpw = NBLOCKS // nw

    mesh = plsc.VectorSubcoreMesh(core_axis_name="c", subcore_axis_name="s")

    scratch = [
        pltpu.VMEM((EMB_DIM * VOCAB,), jnp.float32),
        pltpu.VMEM((bpw, BBLK), jnp.int32),
        pltpu.VMEM((6, ETILES, 8, BBLK), jnp.float32),
        pltpu.SemaphoreType.DMA,
        pltpu.SemaphoreType.DMA,
    ]

    @functools.partial(
        pl.kernel,
        mesh=mesh,
        out_type=jax.ShapeDtypeStruct(
            (HIST, ETILES, NBT, 8, BBLK), jnp.float32),
        scratch_types=scratch,
        compiler_params=pltpu.CompilerParams(
            use_tc_tiling_on_sc=False, needs_layout_passes=False),
    )
    def emb_kernel(idx_hbm, table_hbm, out_hbm, table_v, idx_v, buf, sem_t,
                   sem_o):
        wid = lax.axis_index("s") * nc + lax.axis_index("c")
        base = wid * bpw
        pltpu.async_copy(table_hbm, table_v, sem_t)
        pltpu.sync_copy(idx_hbm.at[wid], idx_v)
        pltpu.make_async_copy(table_hbm, table_v, sem_t).wait()

        ngrp = BBLK // nl

        def block_body(blk, carry):
            p = lax.rem(blk, 6)
            flat = base + blk
            h = flat // NBT
            bt = lax.rem(flat, NBT)

            @pl.when(blk >= 6)
            def _():
                pltpu.make_async_copy(
                    buf.at[0], out_hbm.at[0, pl.ds(0, ETILES), 0],
                    sem_o).wait()

            idxb = [idx_v[blk, pl.ds(nl * j, nl)] for j in range(ngrp)]

            def gathers(e):
                return [
                    plsc.load_gather(table_v, [idxb[j] + (e * TSTRIDE)])
                    for j in range(ngrp)
                ]

            def stores(e, vs):
                for j in range(ngrp):
                    buf[p, e // 8, e % 8, pl.ds(nl * j, nl)] = vs[j]

            vs0 = gathers(0)
            vs1 = gathers(1)
            for e in range(2, EMB_DIM):
                vs2 = []
                for j in range(ngrp):
                    vs2.append(
                        plsc.load_gather(table_v, [idxb[j] + (e * TSTRIDE)]))
                    buf[p, (e - 2) // 8, (e - 2) % 8,
                        pl.ds(nl * j, nl)] = vs0[j]
                vs0, vs1 = vs1, vs2
            stores(EMB_DIM - 2, vs0)
            stores(EMB_DIM - 1, vs1)

            pltpu.async_copy(
                buf.at[p], out_hbm.at[h, pl.ds(0, ETILES), bt], sem_o)
            return carry

        lax.fori_loop(0, bpw, block_body, 0)

        for _ in range(6):
            pltpu.make_async_copy(
                buf.at[0], out_hbm.at[0, pl.ds(0, ETILES), 0],
                sem_o).wait()

    return emb_kernel, nw


_emb_kernel, _NW = _make_kernel()


def kernel(indices, table):
    idxt = indices.T.reshape(_NW, NBLOCKS // _NW, BBLK)
    out5 = _emb_kernel(idxt, table.T.reshape(EMB_DIM * VOCAB))
    return out5.transpose(2, 4, 0, 1, 3).reshape(BATCH, HIST, EMB_DIM)
